# baseline (speedup 1.0000x reference)
import functools

import jax
import jax.numpy as jnp
from jax.experimental import pallas as pl
from jax.experimental.pallas import tpu as pltpu

_B, _S, _D = 2, 2048, 768
_E, _K = 64, 8
_H = 48
_DFF = _E * _H
_TOK_BLK = 1024


def _ffn_body(x_ref, wgt_ref, w1_ref, w2_ref, exp_ref, o_ref):
    xb = x_ref[...]
    g = jnp.dot(xb, wgt_ref[...], preferred_element_type=jnp.float32)
    h = jnp.maximum(
        jax.lax.dot_general(xb.astype(jnp.bfloat16), w1_ref[...],
                            dimension_numbers=(((1,), (1,)), ((), ())),
                            preferred_element_type=jnp.float32), 0.0)
    iota = jax.lax.broadcasted_iota(jnp.int32, g.shape, 1).astype(jnp.float32)
    neg = jnp.float32(jnp.finfo(jnp.float32).min)
    gcur = g
    sel = jnp.zeros(g.shape, dtype=jnp.bool_)
    for _ in range(_K):
        m = jnp.max(gcur, axis=1, keepdims=True)
        eq = gcur == m
        jfirst = jnp.min(jnp.where(eq, iota, jnp.float32(_E)), axis=1,
                         keepdims=True)
        first = iota == jfirst
        sel = sel | first
        gcur = jnp.where(first, neg, gcur)
    mask = sel.astype(jnp.bfloat16)
    mexp = jnp.dot(mask, exp_ref[...], preferred_element_type=jnp.float32)
    o_ref[...] = jnp.dot((h * mexp).astype(jnp.bfloat16), w2_ref[...],
                         preferred_element_type=jnp.float32)


@functools.partial(jax.jit, static_argnames=())
def kernel(x, wg, fc1_w, fc2_w):
    b, s, d = x.shape
    n = b * s
    xf = x.reshape(n, d)
    wgt = wg.T
    w1 = fc1_w.reshape(_DFF, d).astype(jnp.bfloat16)
    w2 = fc2_w.astype(jnp.bfloat16).transpose(0, 2, 1).reshape(_DFF, _D)
    expand = jnp.repeat(jnp.eye(_E, dtype=jnp.bfloat16), _H, axis=1)

    yf = pl.pallas_call(
        _ffn_body,
        grid=(n // _TOK_BLK,),
        in_specs=[
            pl.BlockSpec((_TOK_BLK, d), lambda i: (i, 0)),
            pl.BlockSpec((d, _E), lambda i: (0, 0)),
            pl.BlockSpec((_DFF, d), lambda i: (0, 0)),
            pl.BlockSpec((_DFF, _D), lambda i: (0, 0)),
            pl.BlockSpec((_E, _DFF), lambda i: (0, 0)),
        ],
        out_specs=pl.BlockSpec((_TOK_BLK, _D), lambda i: (i, 0)),
        out_shape=jax.ShapeDtypeStruct((n, _D), jnp.float32),
    )(xf, wgt, w1, w2, expand)
    return yf.reshape(b, s, _D)


# speedup vs baseline: 1.1070x; 1.1070x over previous
import functools

import jax
import jax.numpy as jnp
from jax.experimental import pallas as pl
from jax.experimental.pallas import tpu as pltpu

_B, _S, _D = 2, 2048, 768
_E, _K = 64, 8
_H = 48
_DFF = _E * _H
_TOK_BLK = 1024


def _ffn_body(x_ref, wgt_ref, w1_ref, w2_ref, exp_ref, o_ref):
    xb = x_ref[...]
    g = jnp.dot(xb, wgt_ref[...], preferred_element_type=jnp.float32)
    h = jnp.maximum(
        jax.lax.dot_general(xb, w1_ref[...],
                            dimension_numbers=(((1,), (1,)), ((), ())),
                            preferred_element_type=jnp.float32), 0.0)
    iota = jax.lax.broadcasted_iota(jnp.int32, g.shape, 1).astype(jnp.float32)
    neg = jnp.float32(jnp.finfo(jnp.float32).min)
    gcur = g
    sel = jnp.zeros(g.shape, dtype=jnp.bool_)
    for _ in range(_K):
        m = jnp.max(gcur, axis=1, keepdims=True)
        eq = gcur == m
        jfirst = jnp.min(jnp.where(eq, iota, jnp.float32(_E)), axis=1,
                         keepdims=True)
        first = iota == jfirst
        sel = sel | first
        gcur = jnp.where(first, neg, gcur)
    mask = sel.astype(jnp.bfloat16)
    mexp = jnp.dot(mask, exp_ref[...], preferred_element_type=jnp.float32)
    o_ref[...] = jnp.dot(h * mexp, w2_ref[...],
                         preferred_element_type=jnp.float32)


@functools.partial(jax.jit, static_argnames=())
def kernel(x, wg, fc1_w, fc2_w):
    b, s, d = x.shape
    n = b * s
    xf = x.reshape(n, d)
    wgt = wg.T
    w1 = fc1_w.reshape(_DFF, d)
    w2 = fc2_w.transpose(0, 2, 1).reshape(_DFF, _D)
    expand = jnp.repeat(jnp.eye(_E, dtype=jnp.bfloat16), _H, axis=1)

    yf = pl.pallas_call(
        _ffn_body,
        grid=(n // _TOK_BLK,),
        in_specs=[
            pl.BlockSpec((_TOK_BLK, d), lambda i: (i, 0)),
            pl.BlockSpec((d, _E), lambda i: (0, 0)),
            pl.BlockSpec((_DFF, d), lambda i: (0, 0)),
            pl.BlockSpec((_DFF, _D), lambda i: (0, 0)),
            pl.BlockSpec((_E, _DFF), lambda i: (0, 0)),
        ],
        out_specs=pl.BlockSpec((_TOK_BLK, _D), lambda i: (i, 0)),
        out_shape=jax.ShapeDtypeStruct((n, _D), jnp.float32),
    )(xf, wgt, w1, w2, expand)
    return yf.reshape(b, s, _D)
